# baseline (speedup 1.0000x reference)
import math
import functools

import jax
import jax.numpy as jnp
from jax import lax
from jax.experimental import pallas as pl
from jax.experimental.pallas import tpu as pltpu
from jax.experimental.pallas import tpu_sc as plsc

BQ = 128
NUM_WORKERS = 32
CHUNK_ROWS = 16
LANES = 16


def _counts_sc(idx, s, kn):
    rows_per_w = s // NUM_WORKERS
    mesh = plsc.VectorSubcoreMesh(core_axis_name="c", subcore_axis_name="s")

    @functools.partial(
        pl.kernel,
        out_type=jax.ShapeDtypeStruct((s, s), jnp.float32),
        mesh=mesh,
        scratch_types=[
            pltpu.VMEM((CHUNK_ROWS, kn), jnp.int32),
            pltpu.VMEM((CHUNK_ROWS, s), jnp.float32),
        ],
        compiler_params=pltpu.CompilerParams(needs_layout_passes=False),
    )
    def body(idx_hbm, c_hbm, idx_v, buf_v):
        wid = lax.axis_index("s") * 2 + lax.axis_index("c")
        for chunk in range(rows_per_w // CHUNK_ROWS):
            base = wid * rows_per_w + chunk * CHUNK_ROWS
            pltpu.sync_copy(idx_hbm.at[0, pl.ds(base, CHUNK_ROWS), :], idx_v)

            for r in range(CHUNK_ROWS):
                @plsc.parallel_loop(0, s // LANES, 1, unroll=8)
                def _zero(i):
                    buf_v[r, pl.ds(i * LANES, LANES)] = jnp.zeros(
                        (LANES,), jnp.float32
                    )

            for r in range(CHUNK_ROWS):
                qpos = base + r
                row_ids = jnp.full((LANES,), r, jnp.int32)
                for g in range(kn // LANES):
                    iv = idx_v[r, pl.ds(g * LANES, LANES)]
                    valid = (iv >= 0) & (iv < s) & (iv <= qpos)
                    cnt, last = plsc.scan_count(iv, mask=valid)
                    plsc.addupdate_scatter(
                        buf_v,
                        [row_ids, iv],
                        cnt.astype(jnp.float32),
                        mask=last & valid,
                    )
            pltpu.sync_copy(buf_v, c_hbm.at[pl.ds(base, CHUNK_ROWS), :])

    return body(idx)


def _attn_kernel(c_ref, q_ref, k_ref, v_ref, o_ref,
                 bias_ref, m_ref, den_ref, acc_ref, *, h, s, scale):
    qb = pl.program_id(0)
    jc = pl.program_id(1)
    d = q_ref.shape[-1]

    @pl.when(jc <= qb)
    def _active():
        cc = lax.transpose(c_ref[...], (1, 0))
        bias_ref[...] = jnp.where(cc > 0.0, jnp.log(cc), jnp.float32(-1e30))
        bias = bias_ref[...]

        for hh in range(h):
            q = q_ref[0, hh]
            kc = k_ref[0, hh, pl.ds(jc * BQ, BQ), :]
            vc = v_ref[0, hh, pl.ds(jc * BQ, BQ), :]
            st = lax.dot_general(
                kc, q, (((1,), (1,)), ((), ())),
                preferred_element_type=jnp.float32,
            ) * scale + bias

            if True:
                mx = jnp.max(st, axis=0, keepdims=True)

                @pl.when(jc == 0)
                def _init():
                    e0 = jnp.exp(st - mx)
                    m_ref[hh:hh + 1, :] = mx
                    den_ref[hh:hh + 1, :] = jnp.sum(e0, axis=0, keepdims=True)
                    acc_ref[hh] = lax.dot_general(
                        vc, e0.astype(jnp.bfloat16), (((0,), (0,)), ((), ())),
                        preferred_element_type=jnp.float32,
                    )

                @pl.when(jc > 0)
                def _update():
                    m = m_ref[hh:hh + 1, :]
                    m_new = jnp.maximum(m, mx)
                    alpha = jnp.exp(m - m_new)
                    e = jnp.exp(st - m_new)
                    m_ref[hh:hh + 1, :] = m_new
                    den_ref[hh:hh + 1, :] = (
                        den_ref[hh:hh + 1, :] * alpha
                        + jnp.sum(e, axis=0, keepdims=True)
                    )
                    acc_ref[hh] = acc_ref[hh] * alpha + lax.dot_general(
                        vc, e.astype(jnp.bfloat16), (((0,), (0,)), ((), ())),
                        preferred_element_type=jnp.float32,
                    )

    @pl.when(jc == qb)
    def _finalize():
        for hh in range(h):
            res = acc_ref[hh] / jnp.maximum(den_ref[hh:hh + 1, :], 1e-9)
            o_ref[0, hh] = lax.transpose(res, (1, 0))


@jax.jit
def kernel(q, k, v, neigh_idx):
    b, h, s, d = q.shape
    kn = neigh_idx.shape[-1]
    scale = 1.0 / math.sqrt(d)
    c = _counts_sc(neigh_idx.astype(jnp.int32), s, kn)
    nq = s // BQ
    qb16 = q.astype(jnp.bfloat16)
    kb16 = k.astype(jnp.bfloat16)
    vb16 = v.astype(jnp.bfloat16)

    out = pl.pallas_call(
        functools.partial(_attn_kernel, h=h, s=s, scale=scale),
        grid=(nq, nq),
        in_specs=[
            pl.BlockSpec((BQ, BQ), lambda qb, jc: (qb, jc)),
            pl.BlockSpec((1, h, BQ, d), lambda qb, jc: (0, 0, qb, 0)),
            pl.BlockSpec((1, h, s, d), lambda qb, jc: (0, 0, 0, 0)),
            pl.BlockSpec((1, h, s, d), lambda qb, jc: (0, 0, 0, 0)),
        ],
        out_specs=pl.BlockSpec((1, h, BQ, d), lambda qb, jc: (0, 0, qb, 0)),
        out_shape=jax.ShapeDtypeStruct((b, h, s, d), jnp.float32),
        scratch_shapes=[
            pltpu.VMEM((BQ, BQ), jnp.float32),
            pltpu.VMEM((h, BQ), jnp.float32),
            pltpu.VMEM((h, BQ), jnp.float32),
            pltpu.VMEM((h, d, BQ), jnp.float32),
        ],
    )(c, qb16, kb16, vb16)
    return out


# speedup vs baseline: 2.8810x; 2.8810x over previous
import math
import functools

import jax
import jax.numpy as jnp
from jax import lax
from jax.experimental import pallas as pl
from jax.experimental.pallas import tpu as pltpu
from jax.experimental.pallas import tpu_sc as plsc

BQ = 256
NUM_WORKERS = 32
CHUNK_ROWS = 16
LANES = 16


def _counts_sc(idx, s, kn):
    rows_per_w = s // NUM_WORKERS
    mesh = plsc.VectorSubcoreMesh(core_axis_name="c", subcore_axis_name="s")

    @functools.partial(
        pl.kernel,
        out_type=jax.ShapeDtypeStruct((s, s), jnp.float32),
        mesh=mesh,
        scratch_types=[
            pltpu.VMEM((CHUNK_ROWS, kn), jnp.int32),
            pltpu.VMEM((CHUNK_ROWS, s), jnp.float32),
        ],
        compiler_params=pltpu.CompilerParams(needs_layout_passes=False),
    )
    def body(idx_hbm, c_hbm, idx_v, buf_v):
        wid = lax.axis_index("s") * 2 + lax.axis_index("c")
        for chunk in range(rows_per_w // CHUNK_ROWS):
            base = wid * rows_per_w + chunk * CHUNK_ROWS
            pltpu.sync_copy(idx_hbm.at[0, pl.ds(base, CHUNK_ROWS), :], idx_v)

            for r in range(CHUNK_ROWS):
                @plsc.parallel_loop(0, s // LANES, 1, unroll=8)
                def _zero(i):
                    buf_v[r, pl.ds(i * LANES, LANES)] = jnp.zeros(
                        (LANES,), jnp.float32
                    )

            for r in range(CHUNK_ROWS):
                qpos = base + r
                row_ids = jnp.full((LANES,), r, jnp.int32)
                for g in range(kn // LANES):
                    iv = idx_v[r, pl.ds(g * LANES, LANES)]
                    valid = (iv >= 0) & (iv < s) & (iv <= qpos)
                    cnt, last = plsc.scan_count(iv, mask=valid)
                    plsc.addupdate_scatter(
                        buf_v,
                        [row_ids, iv],
                        cnt.astype(jnp.float32),
                        mask=last & valid,
                    )
            pltpu.sync_copy(buf_v, c_hbm.at[pl.ds(base, CHUNK_ROWS), :])

    return body(idx)


def _attn_kernel(c_ref, q_ref, k_ref, v_ref, o_ref, bias_ref, *state_refs,
                 h, s):
    qb = pl.program_id(0)
    jc = pl.program_id(1)
    d = k_ref.shape[-1]
    md_refs = state_refs[:h]
    acc_refs = state_refs[h:]

    @pl.when(jc <= qb)
    def _active():
        cc = lax.transpose(c_ref[...], (1, 0))
        bias_ref[...] = jnp.where(cc > 0.0, jnp.log(cc), jnp.float32(-1e30))
        bias = bias_ref[...]

        for hh in range(h):
            qt = q_ref[0, hh]
            kc = k_ref[0, hh, pl.ds(jc * BQ, BQ), :]
            vt = v_ref[0, hh, :, pl.ds(jc * BQ, BQ)]
            st = lax.dot_general(
                kc, qt, (((1,), (0,)), ((), ())),
                preferred_element_type=jnp.float32,
            ) + bias
            mx = jnp.max(st, axis=0, keepdims=True)
            md = md_refs[hh]
            acc = acc_refs[hh]

            @pl.when(jc == 0)
            def _init():
                e0 = jnp.exp(st - mx)
                md[0:1, :] = mx
                md[1:2, :] = jnp.sum(e0, axis=0, keepdims=True)
                acc[...] = lax.dot_general(
                    vt, e0.astype(jnp.bfloat16), (((1,), (0,)), ((), ())),
                    preferred_element_type=jnp.float32,
                )

            @pl.when(jc > 0)
            def _update():
                m = md[0:1, :]
                m_new = jnp.maximum(m, mx)
                alpha = jnp.exp(m - m_new)
                e = jnp.exp(st - m_new)
                md[0:1, :] = m_new
                md[1:2, :] = (
                    md[1:2, :] * alpha + jnp.sum(e, axis=0, keepdims=True)
                )
                acc[...] = acc[...] * alpha + lax.dot_general(
                    vt, e.astype(jnp.bfloat16), (((1,), (0,)), ((), ())),
                    preferred_element_type=jnp.float32,
                )

    @pl.when(jc == qb)
    def _finalize():
        for hh in range(h):
            res = acc_refs[hh][...] / jnp.maximum(md_refs[hh][1:2, :], 1e-9)
            o_ref[0, hh] = lax.transpose(res, (1, 0))


@jax.jit
def kernel(q, k, v, neigh_idx):
    b, h, s, d = q.shape
    kn = neigh_idx.shape[-1]
    scale = 1.0 / math.sqrt(d)
    c = _counts_sc(neigh_idx.astype(jnp.int32), s, kn)
    nq = s // BQ
    qt16 = jnp.swapaxes(q * scale, 2, 3).astype(jnp.bfloat16)
    kb16 = k.astype(jnp.bfloat16)
    vt16 = jnp.swapaxes(v, 2, 3).astype(jnp.bfloat16)

    out = pl.pallas_call(
        functools.partial(_attn_kernel, h=h, s=s),
        grid=(nq, nq),
        in_specs=[
            pl.BlockSpec((BQ, BQ), lambda qb, jc: (qb, jc)),
            pl.BlockSpec((1, h, d, BQ), lambda qb, jc: (0, 0, 0, qb)),
            pl.BlockSpec((1, h, s, d), lambda qb, jc: (0, 0, 0, 0)),
            pl.BlockSpec((1, h, d, s), lambda qb, jc: (0, 0, 0, 0)),
        ],
        out_specs=pl.BlockSpec((1, h, BQ, d), lambda qb, jc: (0, 0, qb, 0)),
        out_shape=jax.ShapeDtypeStruct((b, h, s, d), jnp.float32),
        scratch_shapes=(
            [pltpu.VMEM((BQ, BQ), jnp.float32)]
            + [pltpu.VMEM((2, BQ), jnp.float32) for _ in range(h)]
            + [pltpu.VMEM((d, BQ), jnp.float32) for _ in range(h)]
        ),
    )(c, qt16, kb16, vt16)
    return out


# speedup vs baseline: 3.7219x; 1.2919x over previous
import math
import functools

import jax
import jax.numpy as jnp
from jax import lax
from jax.experimental import pallas as pl
from jax.experimental.pallas import tpu as pltpu
from jax.experimental.pallas import tpu_sc as plsc

BQ = 256
NUM_WORKERS = 32
CHUNK_ROWS = 16
LANES = 16


def _counts_sc(idx, s, kn):
    rows_per_w = s // NUM_WORKERS
    mesh = plsc.VectorSubcoreMesh(core_axis_name="c", subcore_axis_name="s")

    @functools.partial(
        pl.kernel,
        out_type=jax.ShapeDtypeStruct((s, s), jnp.float32),
        mesh=mesh,
        scratch_types=[
            pltpu.VMEM((CHUNK_ROWS, kn), jnp.int32),
            pltpu.VMEM((CHUNK_ROWS, s), jnp.float32),
        ],
        compiler_params=pltpu.CompilerParams(needs_layout_passes=False),
    )
    def body(idx_hbm, c_hbm, idx_v, buf_v):
        wid = lax.axis_index("s") * 2 + lax.axis_index("c")
        for chunk in range(rows_per_w // CHUNK_ROWS):
            base = wid * rows_per_w + chunk * CHUNK_ROWS
            pltpu.sync_copy(idx_hbm.at[0, pl.ds(base, CHUNK_ROWS), :], idx_v)

            for r in range(CHUNK_ROWS):
                @plsc.parallel_loop(0, s // LANES, 1, unroll=8)
                def _zero(i):
                    buf_v[r, pl.ds(i * LANES, LANES)] = jnp.zeros(
                        (LANES,), jnp.float32
                    )

            for r in range(CHUNK_ROWS):
                qpos = base + r
                row_ids = jnp.full((LANES,), r, jnp.int32)
                for g in range(kn // LANES):
                    iv = idx_v[r, pl.ds(g * LANES, LANES)]
                    valid = (iv >= 0) & (iv < s) & (iv <= qpos)
                    cnt, last = plsc.scan_count(iv, mask=valid)
                    plsc.addupdate_scatter(
                        buf_v,
                        [row_ids, iv],
                        cnt.astype(jnp.float32),
                        mask=last & valid,
                    )
            pltpu.sync_copy(buf_v, c_hbm.at[pl.ds(base, CHUNK_ROWS), :])

    return body(idx)


def _attn_kernel(c_ref, q_ref, k_ref, v_ref, o_ref, bias_ref, mk_ref,
                 *acc_refs, h, s, kn):
    qb = pl.program_id(0)
    jc = pl.program_id(1)
    d = k_ref.shape[-1]
    lnkn = math.log(float(kn))

    @pl.when((qb == 0) & (jc == 0))
    def _knorms():
        for hh in range(h):
            kf = k_ref[0, hh].astype(jnp.float32)
            n2 = jnp.sum(kf * kf, axis=1, keepdims=True)
            mk = jnp.sqrt(jnp.max(n2))
            mk_ref[hh:hh + 1, :] = jnp.full((1, 128), mk, jnp.float32)

    @pl.when(jc <= qb)
    def _active():
        cc = lax.transpose(c_ref[...], (1, 0))
        bias_ref[...] = jnp.where(cc > 0.0, jnp.log(cc), jnp.float32(-1e30))
        bias = bias_ref[...]

        for hh in range(h):
            qt = q_ref[0, hh]
            qf = qt.astype(jnp.float32)
            qn = jnp.sqrt(jnp.sum(qf * qf, axis=0, keepdims=True))
            mb = qn * mk_ref[hh:hh + 1, 0:1] + lnkn
            kc = k_ref[0, hh, pl.ds(jc * BQ, BQ), :]
            vt = v_ref[0, hh, :, pl.ds(jc * BQ, BQ)]
            st = lax.dot_general(
                kc, qt, (((1,), (0,)), ((), ())),
                preferred_element_type=jnp.float32,
            )
            e = jnp.exp(st + bias - mb).astype(jnp.bfloat16)
            mm = lax.dot_general(
                vt, e, (((1,), (0,)), ((), ())),
                preferred_element_type=jnp.float32,
            )
            acc_refs[hh][...] = jnp.where(jc == 0, mm, acc_refs[hh][...] + mm)

    @pl.when(jc == qb)
    def _finalize():
        for hh in range(h):
            a = acc_refs[hh]
            res = a[0:d, :] / a[d:d + 1, :]
            o_ref[0, hh] = lax.transpose(res, (1, 0))


@jax.jit
def kernel(q, k, v, neigh_idx):
    b, h, s, d = q.shape
    kn = neigh_idx.shape[-1]
    scale = 1.0 / math.sqrt(d)
    c = _counts_sc(neigh_idx.astype(jnp.int32), s, kn)
    nq = s // BQ
    qt16 = jnp.swapaxes(q * scale, 2, 3).astype(jnp.bfloat16)
    kb16 = k.astype(jnp.bfloat16)
    vt = jnp.swapaxes(v, 2, 3)
    dp = 8 * ((d + 1 + 7) // 8)
    pad = jnp.zeros((b, h, dp - d - 1, s), jnp.float32)
    vt16 = jnp.concatenate(
        [vt, jnp.ones((b, h, 1, s), jnp.float32), pad], axis=2
    ).astype(jnp.bfloat16)

    out = pl.pallas_call(
        functools.partial(_attn_kernel, h=h, s=s, kn=kn),
        grid=(nq, nq),
        in_specs=[
            pl.BlockSpec((BQ, BQ), lambda qb, jc: (qb, jc)),
            pl.BlockSpec((1, h, d, BQ), lambda qb, jc: (0, 0, 0, qb)),
            pl.BlockSpec((1, h, s, d), lambda qb, jc: (0, 0, 0, 0)),
            pl.BlockSpec((1, h, dp, s), lambda qb, jc: (0, 0, 0, 0)),
        ],
        out_specs=pl.BlockSpec((1, h, BQ, d), lambda qb, jc: (0, 0, qb, 0)),
        out_shape=jax.ShapeDtypeStruct((b, h, s, d), jnp.float32),
        scratch_shapes=(
            [pltpu.VMEM((BQ, BQ), jnp.float32),
             pltpu.VMEM((h, 128), jnp.float32)]
            + [pltpu.VMEM((dp, BQ), jnp.float32) for _ in range(h)]
        ),
    )(c, qt16, kb16, vt16)
    return out
